# Initial kernel scaffold; baseline (speedup 1.0000x reference)
import jax
import jax.numpy as jnp
from jax.experimental import pallas as pl


def kernel(x, edge_index, batch, W1a, b1a, g1, be1, W1b, b1b, W2a, b2a, g2, be2, W2b, b2b, W3a, b3a, g3, be3, W3b, b3b, lW1, lb1, lW2, lb2):
    raise NotImplementedError("write your pallas kernel here")


# speedup vs baseline: 3.8057x; 3.8057x over previous
import functools

import jax
import jax.numpy as jnp
import numpy as np
from jax import lax
from jax.experimental import pallas as pl
from jax.experimental.pallas import tpu as pltpu
from jax.experimental.pallas import tpu_sc as plsc

N = 10000
E = 160000
F = 256
Q = 64
B = 64
NS = 16
EPT = E // NS
K = 80
NCH = EPT // K
NZC = N // K
INV_SQRT = float(1.0 / np.sqrt(1.0 + 1e-5))


def _sc_agg_body(h0, h1, h2, h3, src3, dst3, out0, out1, out2, out3,
                 src_v, dst_v, g0, g1, acc, sem0, sem1):
    c = lax.axis_index("c")
    s = lax.axis_index("s")

    pltpu.sync_copy(src3.at[s], src_v)
    pltpu.sync_copy(dst3.at[s], dst_v)

    def one_pass(table, out):
        def zstore(t, carry):
            g0[t // 4, pl.ds((t % 4) * 16, 16)] = jnp.zeros((16,), jnp.float32)
            return carry
        lax.fori_loop(0, K * (Q // 16), zstore, 0)

        def zchunk(kk, carry):
            t = s + kk * NS
            @pl.when(t < NZC)
            def _():
                pltpu.sync_copy(g0, acc.at[pl.ds(t * K, K)])
            return carry
        lax.fori_loop(0, (NZC + NS - 1) // NS, zchunk, 0)
        plsc.subcore_barrier()

        pltpu.async_copy(table.at[src_v.at[0]], g0, sem0)
        def body(jj, carry):
            j0 = 2 * jj
            pltpu.make_async_copy(table.at[src_v.at[j0]], g0, sem0).wait()
            pltpu.async_copy(table.at[src_v.at[j0 + 1]], g1, sem1)
            pltpu.sync_copy(g0, acc.at[dst_v.at[j0]], add=True)
            pltpu.make_async_copy(table.at[src_v.at[j0 + 1]], g1, sem1).wait()
            @pl.when(j0 + 2 < NCH)
            def _():
                pltpu.async_copy(table.at[src_v.at[j0 + 2]], g0, sem0)
            pltpu.sync_copy(g1, acc.at[dst_v.at[j0 + 1]], add=True)
            return carry
        lax.fori_loop(0, NCH // 2, body, 0)
        if NCH % 2:
            pltpu.make_async_copy(table.at[src_v.at[NCH - 1]], g0, sem0).wait()
            pltpu.sync_copy(g0, acc.at[dst_v.at[NCH - 1]], add=True)
        plsc.subcore_barrier()

        def cchunk(kk, carry):
            t = s + kk * NS
            @pl.when(t < NZC)
            def _():
                pltpu.sync_copy(acc.at[pl.ds(t * K, K)], out.at[pl.ds(t * K, K)])
            return carry
        lax.fori_loop(0, (NZC + NS - 1) // NS, cchunk, 0)
        plsc.subcore_barrier()

    @pl.when(c == 0)
    def _():
        one_pass(h0, out0)
        one_pass(h1, out1)

    @pl.when(c == 1)
    def _():
        one_pass(h2, out2)
        one_pass(h3, out3)


_sc_agg = functools.partial(
    pl.kernel,
    out_type=tuple(jax.ShapeDtypeStruct((N, Q), jnp.float32)
                   for _ in range(4)),
    mesh=plsc.VectorSubcoreMesh(core_axis_name="c", subcore_axis_name="s"),
    scratch_types=[
        pltpu.VMEM((NCH, K), jnp.int32),
        pltpu.VMEM((NCH, K), jnp.int32),
        pltpu.VMEM((K, Q), jnp.float32),
        pltpu.VMEM((K, Q), jnp.float32),
        pltpu.VMEM_SHARED((N, Q), jnp.float32),
        pltpu.SemaphoreType.DMA,
        pltpu.SemaphoreType.DMA,
    ],
    compiler_params=pltpu.CompilerParams(use_tc_tiling_on_sc=False),
    name="sc_edge_segment_sum",
)(_sc_agg_body)


R = 2000


def _mlp_body(h0_ref, h1_ref, h2_ref, h3_ref, a0_ref, a1_ref, a2_ref, a3_ref,
              Wa_ref, ba_ref, g_ref, be_ref, Wb_ref, bb_ref, batch_ref,
              o0_ref, o1_ref, o2_ref, o3_ref, p_ref):
    z = jnp.concatenate([h0_ref[...] + a0_ref[...],
                         h1_ref[...] + a1_ref[...],
                         h2_ref[...] + a2_ref[...],
                         h3_ref[...] + a3_ref[...]], axis=1)
    z = jnp.dot(z, Wa_ref[...], preferred_element_type=jnp.float32) + ba_ref[...]
    z = z * (g_ref[...] * INV_SQRT) + be_ref[...]
    z = jnp.maximum(z, 0.0)
    z = jnp.dot(z, Wb_ref[...], preferred_element_type=jnp.float32) + bb_ref[...]
    z = jnp.maximum(z, 0.0)
    o0_ref[...] = z[:, 0 * Q:1 * Q]
    o1_ref[...] = z[:, 1 * Q:2 * Q]
    o2_ref[...] = z[:, 2 * Q:3 * Q]
    o3_ref[...] = z[:, 3 * Q:4 * Q]
    onehot = (lax.broadcasted_iota(jnp.int32, (R, B), 1)
              == batch_ref[...]).astype(jnp.float32)
    pblk = lax.dot_general(onehot, z, (((0,), (0,)), ((), ())),
                           preferred_element_type=jnp.float32)
    @pl.when(pl.program_id(0) == 0)
    def _():
        p_ref[...] = jnp.zeros_like(p_ref)
    p_ref[...] += pblk


_mlp = pl.pallas_call(
    _mlp_body,
    grid=(N // R,),
    in_specs=(
        [pl.BlockSpec((R, Q), lambda i: (i, 0)) for _ in range(8)]
        + [
            pl.BlockSpec((F, F), lambda i: (0, 0)),
            pl.BlockSpec((1, F), lambda i: (0, 0)),
            pl.BlockSpec((1, F), lambda i: (0, 0)),
            pl.BlockSpec((1, F), lambda i: (0, 0)),
            pl.BlockSpec((F, F), lambda i: (0, 0)),
            pl.BlockSpec((1, F), lambda i: (0, 0)),
            pl.BlockSpec((R, 1), lambda i: (i, 0)),
        ]
    ),
    out_specs=[pl.BlockSpec((R, Q), lambda i: (i, 0)) for _ in range(4)]
    + [pl.BlockSpec((B, F), lambda i: (0, 0))],
    out_shape=[jax.ShapeDtypeStruct((N, Q), jnp.float32) for _ in range(4)]
    + [jax.ShapeDtypeStruct((B, F), jnp.float32)],
)


def _final_body(p1_ref, p2_ref, p3_ref, lW1_ref, lb1_ref, lW2_ref, lb2_ref,
                out_ref):
    h = jnp.concatenate([p1_ref[...], p2_ref[...], p3_ref[...]], axis=1)
    h = jnp.dot(h, lW1_ref[...], preferred_element_type=jnp.float32) + lb1_ref[...]
    h = jnp.maximum(h, 0.0)
    out_ref[...] = (jnp.dot(h, lW2_ref[...], preferred_element_type=jnp.float32)
                    + lb2_ref[...])


_final = pl.pallas_call(
    _final_body,
    out_shape=jax.ShapeDtypeStruct((B, B), jnp.float32),
)


def kernel(x, edge_index, batch, W1a, b1a, g1, be1, W1b, b1b,
           W2a, b2a, g2, be2, W2b, b2b,
           W3a, b3a, g3, be3, W3b, b3b,
           lW1, lb1, lW2, lb2):
    src3 = edge_index[0].reshape(NS, NCH, K)
    dst3 = edge_index[1].reshape(NS, NCH, K)
    batch2 = batch.reshape(N, 1)
    hq = tuple(x[:, i * Q:(i + 1) * Q] for i in range(4))

    pools = []
    for (Wa, ba, g, be, Wb, bb) in ((W1a, b1a, g1, be1, W1b, b1b),
                                    (W2a, b2a, g2, be2, W2b, b2b),
                                    (W3a, b3a, g3, be3, W3b, b3b)):
        aq = _sc_agg(*hq, src3, dst3)
        *hq, p = _mlp(*hq, *aq, Wa, ba.reshape(1, F), g.reshape(1, F),
                      be.reshape(1, F), Wb, bb.reshape(1, F), batch2)
        pools.append(p)

    return _final(pools[0], pools[1], pools[2], lW1, lb1.reshape(1, 3 * F),
                  lW2, lb2.reshape(1, B))


# speedup vs baseline: 6.2468x; 1.6414x over previous
import functools

import jax
import jax.numpy as jnp
import numpy as np
from jax import lax
from jax.experimental import pallas as pl
from jax.experimental.pallas import tpu as pltpu
from jax.experimental.pallas import tpu_sc as plsc

N = 10000
E = 160000
F = 256
HALF = 128
B = 64
NS = 16
EPT = E // NS
K = 80
NCH = EPT // K
ZK = 80
NZC = N // ZK
INV_SQRT = float(1.0 / np.sqrt(1.0 + 1e-5))


def _sc_agg_body(hL, hR, pk3, outL, outR,
                 pk_v, src_v, dst_v, g0, g1, acc, sg0, sg1, ss0, ss1):
    c = lax.axis_index("c")
    s = lax.axis_index("s")

    pltpu.sync_copy(pk3.at[s], pk_v)
    def unpack(t, carry):
        v = pk_v[pl.ds(t * 16, 16)]
        i = t // (K // 16)
        j = (t % (K // 16)) * 16
        src_v[i, pl.ds(j, 16)] = lax.shift_right_logical(v, 14)
        dst_v[i, pl.ds(j, 16)] = lax.bitwise_and(v, 16383)
        return carry
    lax.fori_loop(0, EPT // 16, unpack, 0)

    def zstore(t, carry):
        g0[t // 8, pl.ds((t % 8) * 16, 16)] = jnp.zeros((16,), jnp.float32)
        return carry
    lax.fori_loop(0, ZK * 8, zstore, 0)

    def zchunk(kk, carry):
        t = s + kk * NS
        @pl.when(t < NZC)
        def _():
            pltpu.sync_copy(g0.at[pl.ds(0, ZK)], acc.at[pl.ds(t * ZK, ZK)])
        return carry
    lax.fori_loop(0, (NZC + NS - 1) // NS, zchunk, 0)
    plsc.subcore_barrier()

    def run(table):
        pltpu.async_copy(table.at[src_v.at[0]], g0, sg0)
        pltpu.async_copy(table.at[src_v.at[1]], g1, sg1)
        def body(jj, carry):
            j0 = 2 * jj
            pltpu.make_async_copy(table.at[src_v.at[j0]], g0, sg0).wait()
            pltpu.async_copy(g0, acc.at[dst_v.at[j0]], ss0, add=True)
            pltpu.make_async_copy(table.at[src_v.at[j0 + 1]], g1, sg1).wait()
            pltpu.async_copy(g1, acc.at[dst_v.at[j0 + 1]], ss1, add=True)
            @pl.when(j0 + 2 < NCH)
            def _():
                pltpu.make_async_copy(g0, acc.at[dst_v.at[j0]], ss0).wait()
                pltpu.async_copy(table.at[src_v.at[j0 + 2]], g0, sg0)
            @pl.when(j0 + 3 < NCH)
            def _():
                pltpu.make_async_copy(g1, acc.at[dst_v.at[j0 + 1]], ss1).wait()
                pltpu.async_copy(table.at[src_v.at[j0 + 3]], g1, sg1)
            return carry
        lax.fori_loop(0, NCH // 2, body, 0)
        if NCH % 2:
            pltpu.make_async_copy(table.at[src_v.at[NCH - 1]], g0, sg0).wait()
            pltpu.async_copy(g0, acc.at[dst_v.at[NCH - 1]], ss0, add=True)
            pltpu.make_async_copy(g0, acc.at[dst_v.at[NCH - 1]], ss0).wait()
            pltpu.make_async_copy(g1, acc.at[dst_v.at[NCH - 2]], ss1).wait()
        else:
            pltpu.make_async_copy(g0, acc.at[dst_v.at[NCH - 2]], ss0).wait()
            pltpu.make_async_copy(g1, acc.at[dst_v.at[NCH - 1]], ss1).wait()

    @pl.when(c == 0)
    def _():
        run(hL)

    @pl.when(c == 1)
    def _():
        run(hR)

    plsc.subcore_barrier()

    def copyout(out):
        def cchunk(kk, carry):
            t = s + kk * NS
            @pl.when(t < NZC)
            def _():
                pltpu.sync_copy(acc.at[pl.ds(t * ZK, ZK)],
                                out.at[pl.ds(t * ZK, ZK)])
            return carry
        lax.fori_loop(0, (NZC + NS - 1) // NS, cchunk, 0)

    @pl.when(c == 0)
    def _():
        copyout(outL)

    @pl.when(c == 1)
    def _():
        copyout(outR)


_sc_agg = functools.partial(
    pl.kernel,
    out_type=tuple(jax.ShapeDtypeStruct((N, HALF), jnp.float32)
                   for _ in range(2)),
    mesh=plsc.VectorSubcoreMesh(core_axis_name="c", subcore_axis_name="s"),
    scratch_types=[
        pltpu.VMEM((EPT,), jnp.int32),
        pltpu.VMEM((NCH, K), jnp.int32),
        pltpu.VMEM((NCH, K), jnp.int32),
        pltpu.VMEM((K, HALF), jnp.float32),
        pltpu.VMEM((K, HALF), jnp.float32),
        pltpu.VMEM_SHARED((N, HALF), jnp.float32),
        pltpu.SemaphoreType.DMA,
        pltpu.SemaphoreType.DMA,
        pltpu.SemaphoreType.DMA,
        pltpu.SemaphoreType.DMA,
    ],
    compiler_params=pltpu.CompilerParams(use_tc_tiling_on_sc=False),
    name="sc_edge_segment_sum",
)(_sc_agg_body)


R = 2000


def _mlp_body(hL_ref, hR_ref, aL_ref, aR_ref, Wa_ref, ba_ref, g_ref, be_ref,
              Wb_ref, bb_ref, batch_ref, oL_ref, oR_ref, p_ref):
    z = jnp.concatenate([hL_ref[...] + aL_ref[...],
                         hR_ref[...] + aR_ref[...]], axis=1)
    z = jnp.dot(z, Wa_ref[...], preferred_element_type=jnp.float32) + ba_ref[...]
    z = z * (g_ref[...] * INV_SQRT) + be_ref[...]
    z = jnp.maximum(z, 0.0)
    z = jnp.dot(z, Wb_ref[...], preferred_element_type=jnp.float32) + bb_ref[...]
    z = jnp.maximum(z, 0.0)
    oL_ref[...] = z[:, :HALF]
    oR_ref[...] = z[:, HALF:]
    onehot = (lax.broadcasted_iota(jnp.int32, (R, B), 1)
              == batch_ref[...]).astype(jnp.float32)
    pblk = lax.dot_general(onehot, z, (((0,), (0,)), ((), ())),
                           preferred_element_type=jnp.float32)
    @pl.when(pl.program_id(0) == 0)
    def _():
        p_ref[...] = jnp.zeros_like(p_ref)
    p_ref[...] += pblk


_mlp = pl.pallas_call(
    _mlp_body,
    grid=(N // R,),
    in_specs=(
        [pl.BlockSpec((R, HALF), lambda i: (i, 0)) for _ in range(4)]
        + [
            pl.BlockSpec((F, F), lambda i: (0, 0)),
            pl.BlockSpec((1, F), lambda i: (0, 0)),
            pl.BlockSpec((1, F), lambda i: (0, 0)),
            pl.BlockSpec((1, F), lambda i: (0, 0)),
            pl.BlockSpec((F, F), lambda i: (0, 0)),
            pl.BlockSpec((1, F), lambda i: (0, 0)),
            pl.BlockSpec((R, 1), lambda i: (i, 0)),
        ]
    ),
    out_specs=[pl.BlockSpec((R, HALF), lambda i: (i, 0)) for _ in range(2)]
    + [pl.BlockSpec((B, F), lambda i: (0, 0))],
    out_shape=[jax.ShapeDtypeStruct((N, HALF), jnp.float32) for _ in range(2)]
    + [jax.ShapeDtypeStruct((B, F), jnp.float32)],
)


def _final_body(p1_ref, p2_ref, p3_ref, lW1_ref, lb1_ref, lW2_ref, lb2_ref,
                out_ref):
    h = jnp.concatenate([p1_ref[...], p2_ref[...], p3_ref[...]], axis=1)
    h = jnp.dot(h, lW1_ref[...], preferred_element_type=jnp.float32) + lb1_ref[...]
    h = jnp.maximum(h, 0.0)
    out_ref[...] = (jnp.dot(h, lW2_ref[...], preferred_element_type=jnp.float32)
                    + lb2_ref[...])


_final = pl.pallas_call(
    _final_body,
    out_shape=jax.ShapeDtypeStruct((B, B), jnp.float32),
)


def kernel(x, edge_index, batch, W1a, b1a, g1, be1, W1b, b1b,
           W2a, b2a, g2, be2, W2b, b2b,
           W3a, b3a, g3, be3, W3b, b3b,
           lW1, lb1, lW2, lb2):
    pk3 = (edge_index[0] * 16384 + edge_index[1]).reshape(NS, EPT)
    batch2 = batch.reshape(N, 1)
    hq = (x[:, :HALF], x[:, HALF:])

    pools = []
    for (Wa, ba, g, be, Wb, bb) in ((W1a, b1a, g1, be1, W1b, b1b),
                                    (W2a, b2a, g2, be2, W2b, b2b),
                                    (W3a, b3a, g3, be3, W3b, b3b)):
        aq = _sc_agg(*hq, pk3)
        *hq, p = _mlp(*hq, *aq, Wa, ba.reshape(1, F), g.reshape(1, F),
                      be.reshape(1, F), Wb, bb.reshape(1, F), batch2)
        hq = tuple(hq)
        pools.append(p)

    return _final(pools[0], pools[1], pools[2], lW1, lb1.reshape(1, 3 * F),
                  lW2, lb2.reshape(1, B))


# speedup vs baseline: 9.4475x; 1.5124x over previous
import functools

import jax
import jax.numpy as jnp
import numpy as np
from jax import lax
from jax.experimental import pallas as pl
from jax.experimental.pallas import tpu as pltpu
from jax.experimental.pallas import tpu_sc as plsc

N = 10000
E = 160000
F = 256
HALF = 128
B = 64
NS = 16
EPT = E // NS
K = 80
NCH = EPT // K
ZK = 80
NZC = N // ZK
INV_SQRT = float(1.0 / np.sqrt(1.0 + 1e-5))


BLK = 25


def _sc_agg_body(hL, hR, pk3, outL, outR,
                 pkblk, s0, s1, s2, s3, d0, d1, d2, d3,
                 g0, g1, g2, g3, acc,
                 sg0, sg1, sg2, sg3, ss0, ss1, ss2, ss3):
    g = (g0, g1, g2, g3)
    sv = (s0, s1, s2, s3)
    dv = (d0, d1, d2, d3)
    sg = (sg0, sg1, sg2, sg3)
    ss = (ss0, ss1, ss2, ss3)
    c = lax.axis_index("c")
    s = lax.axis_index("s")

    def stage_block(jn):
        pltpu.sync_copy(pk3.at[s, pl.ds(jn, BLK)], pkblk)

    def unpack(jn, b):
        r = jn % BLK
        for t in range(K // 16):
            v = pkblk[r, pl.ds(t * 16, 16)]
            sv[b][pl.ds(t * 16, 16)] = lax.shift_right_logical(v, 14)
            dv[b][pl.ds(t * 16, 16)] = lax.bitwise_and(v, 16383)

    def zstore(t, carry):
        g0[t // 8, pl.ds((t % 8) * 16, 16)] = jnp.zeros((16,), jnp.float32)
        return carry
    lax.fori_loop(0, ZK * 8, zstore, 0)

    def zchunk(kk, carry):
        t = s + kk * NS
        @pl.when(t < NZC)
        def _():
            pltpu.sync_copy(g0.at[pl.ds(0, ZK)], acc.at[pl.ds(t * ZK, ZK)])
        return carry
    lax.fori_loop(0, (NZC + NS - 1) // NS, zchunk, 0)
    plsc.subcore_barrier()

    def run(table):
        stage_block(0)
        for b in range(4):
            unpack(b, b)
            pltpu.async_copy(table.at[sv[b]], g[b], sg[b])
        def body(jj, carry):
            j0 = 4 * jj
            for b in range(4):
                j = j0 + b
                pltpu.make_async_copy(table.at[sv[b]], g[b], sg[b]).wait()
                pltpu.async_copy(g[b], acc.at[dv[b]], ss[b], add=True)
                jn = j + 4
                @pl.when(jn < NCH)
                def _(b=b, jn=jn):
                    pltpu.make_async_copy(g[b], acc.at[dv[b]], ss[b]).wait()
                    @pl.when(jn % BLK == 0)
                    def _():
                        stage_block(jn)
                    unpack(jn, b)
                    pltpu.async_copy(table.at[sv[b]], g[b], sg[b])
            return carry
        lax.fori_loop(0, NCH // 4, body, 0)
        pltpu.make_async_copy(table.at[sv[0]], g[0], sg[0]).wait()
        pltpu.async_copy(g[0], acc.at[dv[0]], ss[0], add=True)
        pltpu.make_async_copy(g[0], acc.at[dv[0]], ss[0]).wait()
        for b in range(1, 4):
            pltpu.make_async_copy(g[b], acc.at[dv[b]], ss[b]).wait()

    @pl.when(c == 0)
    def _():
        run(hL)

    @pl.when(c == 1)
    def _():
        run(hR)

    plsc.subcore_barrier()

    def copyout(out):
        def cchunk(kk, carry):
            t = s + kk * NS
            @pl.when(t < NZC)
            def _():
                pltpu.sync_copy(acc.at[pl.ds(t * ZK, ZK)],
                                out.at[pl.ds(t * ZK, ZK)])
            return carry
        lax.fori_loop(0, (NZC + NS - 1) // NS, cchunk, 0)

    @pl.when(c == 0)
    def _():
        copyout(outL)

    @pl.when(c == 1)
    def _():
        copyout(outR)


_sc_agg = functools.partial(
    pl.kernel,
    out_type=tuple(jax.ShapeDtypeStruct((N, HALF), jnp.float32)
                   for _ in range(2)),
    mesh=plsc.VectorSubcoreMesh(core_axis_name="c", subcore_axis_name="s"),
    scratch_types=[pltpu.VMEM((BLK, K), jnp.int32)]
    + [pltpu.VMEM((K,), jnp.int32) for _ in range(8)]
    + [pltpu.VMEM((K, HALF), jnp.float32) for _ in range(4)]
    + [pltpu.VMEM_SHARED((N, HALF), jnp.float32)]
    + [pltpu.SemaphoreType.DMA] * 8,
    compiler_params=pltpu.CompilerParams(use_tc_tiling_on_sc=False),
    name="sc_edge_segment_sum",
)(_sc_agg_body)


R = 2000


def _mlp_body(hL_ref, hR_ref, aL_ref, aR_ref, Wa_ref, ba_ref, g_ref, be_ref,
              Wb_ref, bb_ref, batch_ref, oL_ref, oR_ref, p_ref):
    z = jnp.concatenate([hL_ref[...] + aL_ref[...],
                         hR_ref[...] + aR_ref[...]], axis=1)
    z = jnp.dot(z, Wa_ref[...], preferred_element_type=jnp.float32) + ba_ref[...]
    z = z * (g_ref[...] * INV_SQRT) + be_ref[...]
    z = jnp.maximum(z, 0.0)
    z = jnp.dot(z, Wb_ref[...], preferred_element_type=jnp.float32) + bb_ref[...]
    z = jnp.maximum(z, 0.0)
    oL_ref[...] = z[:, :HALF]
    oR_ref[...] = z[:, HALF:]
    onehot = (lax.broadcasted_iota(jnp.int32, (R, B), 1)
              == batch_ref[...]).astype(jnp.float32)
    pblk = lax.dot_general(onehot, z, (((0,), (0,)), ((), ())),
                           preferred_element_type=jnp.float32)
    @pl.when(pl.program_id(0) == 0)
    def _():
        p_ref[...] = jnp.zeros_like(p_ref)
    p_ref[...] += pblk


_mlp = pl.pallas_call(
    _mlp_body,
    grid=(N // R,),
    in_specs=(
        [pl.BlockSpec((R, HALF), lambda i: (i, 0)) for _ in range(4)]
        + [
            pl.BlockSpec((F, F), lambda i: (0, 0)),
            pl.BlockSpec((1, F), lambda i: (0, 0)),
            pl.BlockSpec((1, F), lambda i: (0, 0)),
            pl.BlockSpec((1, F), lambda i: (0, 0)),
            pl.BlockSpec((F, F), lambda i: (0, 0)),
            pl.BlockSpec((1, F), lambda i: (0, 0)),
            pl.BlockSpec((R, 1), lambda i: (i, 0)),
        ]
    ),
    out_specs=[pl.BlockSpec((R, HALF), lambda i: (i, 0)) for _ in range(2)]
    + [pl.BlockSpec((B, F), lambda i: (0, 0))],
    out_shape=[jax.ShapeDtypeStruct((N, HALF), jnp.float32) for _ in range(2)]
    + [jax.ShapeDtypeStruct((B, F), jnp.float32)],
)


def _final_body(p1_ref, p2_ref, p3_ref, lW1_ref, lb1_ref, lW2_ref, lb2_ref,
                out_ref):
    h = jnp.concatenate([p1_ref[...], p2_ref[...], p3_ref[...]], axis=1)
    h = jnp.dot(h, lW1_ref[...], preferred_element_type=jnp.float32) + lb1_ref[...]
    h = jnp.maximum(h, 0.0)
    out_ref[...] = (jnp.dot(h, lW2_ref[...], preferred_element_type=jnp.float32)
                    + lb2_ref[...])


_final = pl.pallas_call(
    _final_body,
    out_shape=jax.ShapeDtypeStruct((B, B), jnp.float32),
)


def kernel(x, edge_index, batch, W1a, b1a, g1, be1, W1b, b1b,
           W2a, b2a, g2, be2, W2b, b2b,
           W3a, b3a, g3, be3, W3b, b3b,
           lW1, lb1, lW2, lb2):
    pk3 = (edge_index[0] * 16384 + edge_index[1]).reshape(NS, NCH, K)
    batch2 = batch.reshape(N, 1)
    hq = (x[:, :HALF], x[:, HALF:])

    pools = []
    for (Wa, ba, g, be, Wb, bb) in ((W1a, b1a, g1, be1, W1b, b1b),
                                    (W2a, b2a, g2, be2, W2b, b2b),
                                    (W3a, b3a, g3, be3, W3b, b3b)):
        aq = _sc_agg(*hq, pk3)
        *hq, p = _mlp(*hq, *aq, Wa, ba.reshape(1, F), g.reshape(1, F),
                      be.reshape(1, F), Wb, bb.reshape(1, F), batch2)
        hq = tuple(hq)
        pools.append(p)

    return _final(pools[0], pools[1], pools[2], lW1, lb1.reshape(1, 3 * F),
                  lW2, lb2.reshape(1, B))
